# baseline (speedup 1.0000x reference)
import functools

import jax
import jax.numpy as jnp
from jax import lax
from jax.experimental import pallas as pl
from jax.experimental.pallas import tpu as pltpu
from jax.experimental.pallas import tpu_sc as plsc

VOCAB = 100000
HIDDEN = 768
S_LEN = 512
TT_VOCAB = 2
B_SZ = 64
N_TOK = B_SZ * S_LEN
LN_EPS = 1e-12

NW = 32
TPW = N_TOK // NW
CHUNK = 64
NCH = TPW // CHUNK
HS = HIDDEN // 16

_mesh = plsc.VectorSubcoreMesh(core_axis_name="c", subcore_axis_name="s")
_cparams = pltpu.CompilerParams(needs_layout_passes=False)

_BROWS_PW = (TT_VOCAB * S_LEN) // NW


@functools.partial(
    pl.kernel,
    mesh=_mesh,
    out_type=jax.ShapeDtypeStruct((TT_VOCAB * S_LEN, HIDDEN), jnp.float32),
    compiler_params=_cparams,
    scratch_types=[
        pltpu.VMEM((_BROWS_PW, HIDDEN), jnp.float32),
        pltpu.VMEM((HIDDEN,), jnp.float32),
    ],
)
def _build_bias_tab(tt_hbm, pos_hbm, out_hbm, pos_v, tt_v):
    wid = lax.axis_index("s") * 2 + lax.axis_index("c")
    tt = wid // (NW // TT_VOCAB)
    prow = (wid % (NW // TT_VOCAB)) * _BROWS_PW
    pltpu.sync_copy(pos_hbm.at[pl.ds(prow, _BROWS_PW)], pos_v)
    pltpu.sync_copy(tt_hbm.at[tt], tt_v)

    def body(i, _):
        r = i // HS
        hs = pl.ds((i % HS) * 16, 16)
        pos_v[r, hs] = pos_v[r, hs] + tt_v[hs]
        return 0

    lax.fori_loop(0, _BROWS_PW * HS, body, 0)
    pltpu.sync_copy(pos_v, out_hbm.at[pl.ds(wid * _BROWS_PW, _BROWS_PW)])


@functools.partial(
    pl.kernel,
    mesh=_mesh,
    out_type=jax.ShapeDtypeStruct((N_TOK, HIDDEN), jnp.float32),
    compiler_params=_cparams,
    scratch_types=[
        pltpu.VMEM((NCH, CHUNK), jnp.int32),
        pltpu.VMEM((NCH, CHUNK), jnp.int32),
        pltpu.VMEM((CHUNK, HIDDEN), jnp.float32),
        pltpu.VMEM((CHUNK, HIDDEN), jnp.float32),
        pltpu.VMEM((HIDDEN,), jnp.float32),
        pltpu.VMEM((HIDDEN,), jnp.float32),
        pltpu.VMEM((16,), jnp.float32),
        pltpu.SemaphoreType.DMA,
        pltpu.SemaphoreType.DMA,
    ],
)
def _emb_layernorm(ids_hbm, bidx_hbm, word_hbm, btab_hbm, gam_hbm, bet_hbm,
                   out_hbm, widx_v, bidx_v, rows_v, bias_v, gam_v, bet_v,
                   red_v, sem_w, sem_b):
    wid = lax.axis_index("s") * 2 + lax.axis_index("c")
    base = wid * TPW
    pltpu.sync_copy(ids_hbm.at[wid], widx_v)
    pltpu.sync_copy(bidx_hbm.at[wid], bidx_v)
    pltpu.sync_copy(gam_hbm, gam_v)
    pltpu.sync_copy(bet_hbm, bet_v)

    def chunk_body(c, _):
        cp_w = pltpu.async_copy(word_hbm.at[widx_v.at[c]], rows_v, sem_w)
        cp_b = pltpu.async_copy(btab_hbm.at[bidx_v.at[c]], bias_v, sem_b)
        cp_w.wait()
        cp_b.wait()

        def tok_body(t, _):
            def pass1(h, acc):
                acc_s, acc_q = acc
                hs = pl.ds(h * 16, 16)
                v = rows_v[t, hs] + bias_v[t, hs]
                rows_v[t, hs] = v
                return (acc_s + v, acc_q + v * v)

            zero = jnp.zeros((16,), jnp.float32)
            acc_s, acc_q = lax.fori_loop(0, HS, pass1, (zero, zero))

            def lanesum(x):
                for k in (8, 4, 2, 1):
                    red_v[pl.ds(0, 16)] = x
                    idx = lax.iota(jnp.int32, 16) ^ k
                    x = x + plsc.load_gather(red_v, [idx])
                return x

            mv = lanesum(acc_s) * (1.0 / HIDDEN)
            xv = lanesum(acc_q) * (1.0 / HIDDEN) - mv * mv + LN_EPS
            iv = plsc.bitcast(xv, jnp.int32)
            iv = 0x5F3759DF - lax.shift_right_logical(iv, 1)
            y = plsc.bitcast(iv, jnp.float32)
            xh = xv * 0.5
            y = y * (1.5 - xh * y * y)
            y = y * (1.5 - xh * y * y)
            y = y * (1.5 - xh * y * y)

            def pass2(h, _):
                hs = pl.ds(h * 16, 16)
                rows_v[t, hs] = (rows_v[t, hs] - mv) * y * gam_v[hs] + bet_v[hs]
                return 0

            lax.fori_loop(0, HS, pass2, 0)
            return 0

        lax.fori_loop(0, CHUNK, tok_body, 0)
        pltpu.sync_copy(rows_v, out_hbm.at[pl.ds(base + c * CHUNK, CHUNK)])
        return 0

    lax.fori_loop(0, NCH, chunk_body, 0)


def kernel(input_ids, token_type_ids, word_emb, token_type_emb, pos_emb,
           ln_gamma, ln_beta):
    btab = _build_bias_tab(token_type_emb, pos_emb[:S_LEN])
    ids = input_ids.reshape(NW, NCH, CHUNK)
    bidx = (token_type_ids * S_LEN
            + jnp.arange(S_LEN, dtype=jnp.int32)[None, :]).reshape(NW, NCH, CHUNK)
    out = _emb_layernorm(ids, bidx, word_emb, btab, ln_gamma, ln_beta)
    return out.reshape(B_SZ, S_LEN, HIDDEN)


# speedup vs baseline: 1.2045x; 1.2045x over previous
import functools

import jax
import jax.numpy as jnp
from jax import lax
from jax.experimental import pallas as pl
from jax.experimental.pallas import tpu as pltpu
from jax.experimental.pallas import tpu_sc as plsc

VOCAB = 100000
HIDDEN = 768
S_LEN = 512
TT_VOCAB = 2
B_SZ = 64
N_TOK = B_SZ * S_LEN
LN_EPS = 1e-12

NW = 32
TPW = N_TOK // NW
CHUNK = 16
NCH = TPW // CHUNK
HS = HIDDEN // 16

_mesh = plsc.VectorSubcoreMesh(core_axis_name="c", subcore_axis_name="s")
_cparams = pltpu.CompilerParams(needs_layout_passes=False)

_BROWS_PW = (TT_VOCAB * S_LEN) // NW


@functools.partial(
    pl.kernel,
    mesh=_mesh,
    out_type=jax.ShapeDtypeStruct((TT_VOCAB * S_LEN, HIDDEN), jnp.float32),
    compiler_params=_cparams,
    scratch_types=[
        pltpu.VMEM((_BROWS_PW, HIDDEN), jnp.float32),
        pltpu.VMEM((HIDDEN,), jnp.float32),
    ],
)
def _build_bias_tab(tt_hbm, pos_hbm, out_hbm, pos_v, tt_v):
    wid = lax.axis_index("s") * 2 + lax.axis_index("c")
    tt = wid // (NW // TT_VOCAB)
    prow = (wid % (NW // TT_VOCAB)) * _BROWS_PW
    pltpu.sync_copy(pos_hbm.at[pl.ds(prow, _BROWS_PW)], pos_v)
    pltpu.sync_copy(tt_hbm.at[tt], tt_v)

    def body(i, _):
        r = i // HS
        hs = pl.ds((i % HS) * 16, 16)
        pos_v[r, hs] = pos_v[r, hs] + tt_v[hs]
        return 0

    lax.fori_loop(0, _BROWS_PW * HS, body, 0)
    pltpu.sync_copy(pos_v, out_hbm.at[pl.ds(wid * _BROWS_PW, _BROWS_PW)])


@functools.partial(
    pl.kernel,
    mesh=_mesh,
    out_type=jax.ShapeDtypeStruct((N_TOK, HIDDEN), jnp.float32),
    compiler_params=_cparams,
    scratch_types=[
        pltpu.VMEM((NCH, CHUNK), jnp.int32),
        pltpu.VMEM((NCH, CHUNK), jnp.int32),
        pltpu.VMEM((CHUNK, HIDDEN), jnp.float32),
        pltpu.VMEM((CHUNK, HIDDEN), jnp.float32),
        pltpu.VMEM((CHUNK, HIDDEN), jnp.float32),
        pltpu.VMEM((CHUNK, HIDDEN), jnp.float32),
        pltpu.VMEM((CHUNK, HIDDEN), jnp.float32),
        pltpu.VMEM((CHUNK, HIDDEN), jnp.float32),
        pltpu.VMEM((HIDDEN,), jnp.float32),
        pltpu.VMEM((HIDDEN,), jnp.float32),
        pltpu.VMEM((CHUNK * 16,), jnp.float32),
        pltpu.VMEM((CHUNK * 16,), jnp.float32),
        pltpu.VMEM((CHUNK,), jnp.float32),
        pltpu.VMEM((CHUNK,), jnp.float32),
        pltpu.SemaphoreType.DMA,
        pltpu.SemaphoreType.DMA,
        pltpu.SemaphoreType.DMA,
        pltpu.SemaphoreType.DMA,
        pltpu.SemaphoreType.DMA,
        pltpu.SemaphoreType.DMA,
    ],
)
def _emb_layernorm(ids_hbm, bidx_hbm, word_hbm, btab_hbm, gam_hbm, bet_hbm,
                   out_hbm, widx_v, bidx_v, rows0_v, rows1_v, bias0_v, bias1_v,
                   outb0_v, outb1_v, gam_v, bet_v, sums_v, sq_v, a_v, b_v,
                   sem_w0, sem_b0, sem_w1, sem_b1, sem_o0, sem_o1):
    wid = lax.axis_index("s") * 2 + lax.axis_index("c")
    wbase = wid * TPW
    pltpu.sync_copy(ids_hbm.at[wid], widx_v)
    pltpu.sync_copy(bidx_hbm.at[wid], bidx_v)
    pltpu.sync_copy(gam_hbm, gam_v)
    pltpu.sync_copy(bet_hbm, bet_v)

    bufs = ((rows0_v, bias0_v, outb0_v, sem_w0, sem_b0, sem_o0),
            (rows1_v, bias1_v, outb1_v, sem_w1, sem_b1, sem_o1))

    def issue(ci, rows, bias, sw, sb):
        pltpu.async_copy(word_hbm.at[widx_v.at[ci]], rows, sw)
        pltpu.async_copy(btab_hbm.at[bidx_v.at[ci]], bias, sb)

    def wait_gathers(ci, rows, bias, sw, sb):
        pltpu.make_async_copy(word_hbm.at[widx_v.at[ci]], rows, sw).wait()
        pltpu.make_async_copy(btab_hbm.at[bidx_v.at[ci]], bias, sb).wait()

    def compute(ci, rows, bias, outb):
        def p1(t, _):
            def body(i, acc):
                acc_s, acc_q = acc
                for j in range(4):
                    hs = pl.ds((i * 4 + j) * 16, 16)
                    v = rows[t, hs] + bias[t, hs]
                    rows[t, hs] = v
                    acc_s = acc_s + v
                    acc_q = acc_q + v * v
                return (acc_s, acc_q)

            z = jnp.zeros((16,), jnp.float32)
            acc_s, acc_q = lax.fori_loop(0, HS // 4, body, (z, z))
            sums_v[pl.ds(t * 16, 16)] = acc_s
            sq_v[pl.ds(t * 16, 16)] = acc_q
            return 0

        lax.fori_loop(0, CHUNK, p1, 0)

        col = lax.iota(jnp.int32, 16) * 16
        s_tot = jnp.zeros((16,), jnp.float32)
        q_tot = jnp.zeros((16,), jnp.float32)
        for l in range(16):
            s_tot = s_tot + plsc.load_gather(sums_v, [col + l])
            q_tot = q_tot + plsc.load_gather(sq_v, [col + l])
        mean = s_tot * (1.0 / HIDDEN)
        x = q_tot * (1.0 / HIDDEN) - mean * mean + LN_EPS
        iv = plsc.bitcast(x, jnp.int32)
        iv = 0x5F3759DF - lax.shift_right_logical(iv, 1)
        y = plsc.bitcast(iv, jnp.float32)
        xh = x * 0.5
        y = y * (1.5 - xh * y * y)
        y = y * (1.5 - xh * y * y)
        y = y * (1.5 - xh * y * y)
        a_v[pl.ds(0, CHUNK)] = y
        b_v[pl.ds(0, CHUNK)] = mean * y

        def p3(t, _):
            ti = jnp.full((16,), t, jnp.int32)
            a = plsc.load_gather(a_v, [ti])
            b = plsc.load_gather(b_v, [ti])

            def body(i, _):
                for j in range(4):
                    hs = pl.ds((i * 4 + j) * 16, 16)
                    v = rows[t, hs]
                    outb[t, hs] = (v * a - b) * gam_v[hs] + bet_v[hs]
                return 0

            lax.fori_loop(0, HS // 4, body, 0)
            return 0

        lax.fori_loop(0, CHUNK, p3, 0)

    issue(0, rows0_v, bias0_v, sem_w0, sem_b0)
    issue(1, rows1_v, bias1_v, sem_w1, sem_b1)

    def pair(c, _):
        for k in (0, 1):
            rows, bias, outb, sw, sb, so = bufs[k]
            ci = 2 * c + k

            wait_gathers(ci, rows, bias, sw, sb)

            @pl.when(ci >= 2)
            def _():
                pltpu.make_async_copy(
                    outb, out_hbm.at[pl.ds(wbase, CHUNK)], so).wait()

            compute(ci, rows, bias, outb)
            pltpu.async_copy(
                outb, out_hbm.at[pl.ds(wbase + ci * CHUNK, CHUNK)], so)

            @pl.when(ci + 2 < NCH)
            def _():
                issue(ci + 2, rows, bias, sw, sb)
        return 0

    lax.fori_loop(0, NCH // 2, pair, 0)

    pltpu.make_async_copy(outb0_v, out_hbm.at[pl.ds(wbase, CHUNK)], sem_o0).wait()
    pltpu.make_async_copy(outb1_v, out_hbm.at[pl.ds(wbase, CHUNK)], sem_o1).wait()


def kernel(input_ids, token_type_ids, word_emb, token_type_emb, pos_emb,
           ln_gamma, ln_beta):
    btab = _build_bias_tab(token_type_emb, pos_emb[:S_LEN])
    ids = input_ids.reshape(NW, NCH, CHUNK)
    bidx = (token_type_ids * S_LEN
            + jnp.arange(S_LEN, dtype=jnp.int32)[None, :]).reshape(NW, NCH, CHUNK)
    out = _emb_layernorm(ids, bidx, word_emb, btab, ln_gamma, ln_beta)
    return out.reshape(B_SZ, S_LEN, HIDDEN)


# speedup vs baseline: 4.5135x; 3.7472x over previous
import functools

import jax
import jax.numpy as jnp
from jax import lax
from jax.experimental import pallas as pl
from jax.experimental.pallas import tpu as pltpu
from jax.experimental.pallas import tpu_sc as plsc

VOCAB = 100000
HIDDEN = 768
S_LEN = 512
TT_VOCAB = 2
B_SZ = 64
N_TOK = B_SZ * S_LEN
LN_EPS = 1e-12

NW = 32
TPW = N_TOK // NW
CHUNK = 16
NCH = TPW // CHUNK
HS = HIDDEN // 16

_mesh = plsc.VectorSubcoreMesh(core_axis_name="c", subcore_axis_name="s")
_cparams = pltpu.CompilerParams(needs_layout_passes=False)

_BROWS_PW = (TT_VOCAB * S_LEN) // NW


@functools.partial(
    pl.kernel,
    mesh=_mesh,
    out_type=jax.ShapeDtypeStruct((TT_VOCAB * S_LEN, HIDDEN), jnp.float32),
    compiler_params=_cparams,
    scratch_types=[
        pltpu.VMEM((_BROWS_PW, HIDDEN), jnp.float32),
        pltpu.VMEM((HIDDEN,), jnp.float32),
    ],
)
def _build_bias_tab(tt_hbm, pos_hbm, out_hbm, pos_v, tt_v):
    wid = lax.axis_index("s") * 2 + lax.axis_index("c")
    tt = wid // (NW // TT_VOCAB)
    prow = (wid % (NW // TT_VOCAB)) * _BROWS_PW
    pltpu.sync_copy(pos_hbm.at[pl.ds(prow, _BROWS_PW)], pos_v)
    pltpu.sync_copy(tt_hbm.at[tt], tt_v)

    def body(i, _):
        r = i // HS
        hs = pl.ds((i % HS) * 16, 16)
        pos_v[r, hs] = pos_v[r, hs] + tt_v[hs]
        return 0

    lax.fori_loop(0, _BROWS_PW * HS, body, 0)
    pltpu.sync_copy(pos_v, out_hbm.at[pl.ds(wid * _BROWS_PW, _BROWS_PW)])


@functools.partial(
    pl.kernel,
    mesh=_mesh,
    out_type=jax.ShapeDtypeStruct((N_TOK, HIDDEN), jnp.float32),
    compiler_params=_cparams,
    scratch_types=[
        pltpu.VMEM((NCH, CHUNK), jnp.int32),
        pltpu.VMEM((NCH, CHUNK), jnp.int32),
        pltpu.VMEM((CHUNK, HIDDEN), jnp.float32),
        pltpu.VMEM((CHUNK, HIDDEN), jnp.float32),
        pltpu.VMEM((CHUNK, HIDDEN), jnp.float32),
        pltpu.VMEM((CHUNK, HIDDEN), jnp.float32),
        pltpu.VMEM((CHUNK, HIDDEN), jnp.float32),
        pltpu.VMEM((CHUNK, HIDDEN), jnp.float32),
        pltpu.VMEM((HIDDEN,), jnp.float32),
        pltpu.VMEM((HIDDEN,), jnp.float32),
        pltpu.VMEM((CHUNK * 16,), jnp.float32),
        pltpu.VMEM((CHUNK * 16,), jnp.float32),
        pltpu.VMEM((CHUNK,), jnp.float32),
        pltpu.VMEM((CHUNK,), jnp.float32),
        pltpu.SemaphoreType.DMA,
        pltpu.SemaphoreType.DMA,
        pltpu.SemaphoreType.DMA,
        pltpu.SemaphoreType.DMA,
        pltpu.SemaphoreType.DMA,
        pltpu.SemaphoreType.DMA,
    ],
)
def _emb_layernorm(ids_hbm, bidx_hbm, word_hbm, btab_hbm, gam_hbm, bet_hbm,
                   out_hbm, widx_v, bidx_v, rows0_v, rows1_v, bias0_v, bias1_v,
                   outb0_v, outb1_v, gam_v, bet_v, sums_v, sq_v, a_v, b_v,
                   sem_w0, sem_b0, sem_w1, sem_b1, sem_o0, sem_o1):
    wid = lax.axis_index("s") * 2 + lax.axis_index("c")
    wbase = wid * TPW
    pltpu.sync_copy(ids_hbm.at[wid], widx_v)
    pltpu.sync_copy(bidx_hbm.at[wid], bidx_v)
    pltpu.sync_copy(gam_hbm, gam_v)
    pltpu.sync_copy(bet_hbm, bet_v)

    bufs = ((rows0_v, bias0_v, outb0_v, sem_w0, sem_b0, sem_o0),
            (rows1_v, bias1_v, outb1_v, sem_w1, sem_b1, sem_o1))

    def issue(ci, rows, bias, sw, sb):
        pltpu.async_copy(word_hbm.at[widx_v.at[ci]], rows, sw)
        pltpu.async_copy(btab_hbm.at[bidx_v.at[ci]], bias, sb)

    def wait_gathers(ci, rows, bias, sw, sb):
        pltpu.make_async_copy(word_hbm.at[widx_v.at[ci]], rows, sw).wait()
        pltpu.make_async_copy(btab_hbm.at[bidx_v.at[ci]], bias, sb).wait()

    def compute(ci, rows, bias, outb):
        @plsc.parallel_loop(0, CHUNK)
        def p1(t):
            z = jnp.zeros((16,), jnp.float32)

            @plsc.parallel_loop(0, HS, unroll=8, carry=(z, z))
            def accs(h, acc):
                acc_s, acc_q = acc
                hs = pl.ds(h * 16, 16)
                v = rows[t, hs] + bias[t, hs]
                rows[t, hs] = v
                return (acc_s + v, acc_q + v * v)

            acc_s, acc_q = accs
            sums_v[pl.ds(t * 16, 16)] = acc_s
            sq_v[pl.ds(t * 16, 16)] = acc_q

        col = lax.iota(jnp.int32, 16) * 16
        s_tot = jnp.zeros((16,), jnp.float32)
        q_tot = jnp.zeros((16,), jnp.float32)
        for l in range(16):
            s_tot = s_tot + plsc.load_gather(sums_v, [col + l])
            q_tot = q_tot + plsc.load_gather(sq_v, [col + l])
        mean = s_tot * (1.0 / HIDDEN)
        x = q_tot * (1.0 / HIDDEN) - mean * mean + LN_EPS
        iv = plsc.bitcast(x, jnp.int32)
        iv = 0x5F3759DF - lax.shift_right_logical(iv, 1)
        y = plsc.bitcast(iv, jnp.float32)
        xh = x * 0.5
        y = y * (1.5 - xh * y * y)
        y = y * (1.5 - xh * y * y)
        y = y * (1.5 - xh * y * y)
        a_v[pl.ds(0, CHUNK)] = y
        b_v[pl.ds(0, CHUNK)] = mean * y

        @plsc.parallel_loop(0, CHUNK // 4)
        def p3(g):
            t0 = g * 4
            ab = []
            for j in range(4):
                ti = jnp.full((16,), t0 + j, jnp.int32)
                ab.append((plsc.load_gather(a_v, [ti]),
                           plsc.load_gather(b_v, [ti])))

            @plsc.parallel_loop(0, HS, unroll=4)
            def apply(h):
                hs = pl.ds(h * 16, 16)
                gm = gam_v[hs]
                bt = bet_v[hs]
                for j in range(4):
                    a, b = ab[j]
                    outb[t0 + j, hs] = (rows[t0 + j, hs] * a - b) * gm + bt

    issue(0, rows0_v, bias0_v, sem_w0, sem_b0)
    issue(1, rows1_v, bias1_v, sem_w1, sem_b1)

    def pair(c, _):
        for k in (0, 1):
            rows, bias, outb, sw, sb, so = bufs[k]
            ci = 2 * c + k

            wait_gathers(ci, rows, bias, sw, sb)

            @pl.when(ci >= 2)
            def _():
                pltpu.make_async_copy(
                    outb, out_hbm.at[pl.ds(wbase, CHUNK)], so).wait()

            compute(ci, rows, bias, outb)
            pltpu.async_copy(
                outb, out_hbm.at[pl.ds(wbase + ci * CHUNK, CHUNK)], so)

            @pl.when(ci + 2 < NCH)
            def _():
                issue(ci + 2, rows, bias, sw, sb)
        return 0

    lax.fori_loop(0, NCH // 2, pair, 0)

    pltpu.make_async_copy(outb0_v, out_hbm.at[pl.ds(wbase, CHUNK)], sem_o0).wait()
    pltpu.make_async_copy(outb1_v, out_hbm.at[pl.ds(wbase, CHUNK)], sem_o1).wait()


def kernel(input_ids, token_type_ids, word_emb, token_type_emb, pos_emb,
           ln_gamma, ln_beta):
    btab = _build_bias_tab(token_type_emb, pos_emb[:S_LEN])
    ids = input_ids.reshape(NW, NCH, CHUNK)
    bidx = (token_type_ids * S_LEN
            + jnp.arange(S_LEN, dtype=jnp.int32)[None, :]).reshape(NW, NCH, CHUNK)
    out = _emb_layernorm(ids, bidx, word_emb, btab, ln_gamma, ln_beta)
    return out.reshape(B_SZ, S_LEN, HIDDEN)


# speedup vs baseline: 5.2807x; 1.1700x over previous
import functools

import jax
import jax.numpy as jnp
from jax import lax
from jax.experimental import pallas as pl
from jax.experimental.pallas import tpu as pltpu
from jax.experimental.pallas import tpu_sc as plsc

VOCAB = 100000
HIDDEN = 768
S_LEN = 512
TT_VOCAB = 2
B_SZ = 64
N_TOK = B_SZ * S_LEN
LN_EPS = 1e-12

NW = 32
SP_W = S_LEN // NW
CHUNK = 16
NCH = B_SZ
HS = HIDDEN // 16
NBROWS = TT_VOCAB * SP_W

_mesh = plsc.VectorSubcoreMesh(core_axis_name="c", subcore_axis_name="s")
_cparams = pltpu.CompilerParams(needs_layout_passes=False)


@functools.partial(
    pl.kernel,
    mesh=_mesh,
    out_type=jax.ShapeDtypeStruct((N_TOK, HIDDEN), jnp.float32),
    compiler_params=_cparams,
    scratch_types=[
        pltpu.VMEM((NCH, CHUNK), jnp.int32),
        pltpu.VMEM((NCH, CHUNK), jnp.int32),
        pltpu.VMEM((CHUNK, HIDDEN), jnp.float32),
        pltpu.VMEM((CHUNK, HIDDEN), jnp.float32),
        pltpu.VMEM((CHUNK, HIDDEN), jnp.float32),
        pltpu.VMEM((CHUNK, HIDDEN), jnp.float32),
        pltpu.VMEM((NBROWS, HIDDEN), jnp.float32),
        pltpu.VMEM((TT_VOCAB, HIDDEN), jnp.float32),
        pltpu.VMEM((HIDDEN,), jnp.float32),
        pltpu.VMEM((HIDDEN,), jnp.float32),
        pltpu.VMEM((CHUNK * 16,), jnp.float32),
        pltpu.VMEM((CHUNK * 16,), jnp.float32),
        pltpu.VMEM((CHUNK,), jnp.float32),
        pltpu.VMEM((CHUNK,), jnp.float32),
        pltpu.SemaphoreType.DMA,
        pltpu.SemaphoreType.DMA,
        pltpu.SemaphoreType.DMA,
        pltpu.SemaphoreType.DMA,
    ],
)
def _emb_layernorm(ids_hbm, brow_hbm, word_hbm, tt_hbm, pos_hbm, gam_hbm,
                   bet_hbm, out_hbm, widx_v, brow_v, rows0_v, rows1_v,
                   outb0_v, outb1_v, ttb_v, ttbuf_v, gam_v, bet_v, sums_v,
                   sq_v, a_v, b_v, sem_w0, sem_w1, sem_o0, sem_o1):
    wid = lax.axis_index("s") * 2 + lax.axis_index("c")
    pltpu.sync_copy(ids_hbm.at[wid], widx_v)
    pltpu.sync_copy(brow_hbm.at[wid], brow_v)
    pltpu.sync_copy(gam_hbm, gam_v)
    pltpu.sync_copy(bet_hbm, bet_v)
    pltpu.sync_copy(tt_hbm, ttbuf_v)
    pltpu.sync_copy(pos_hbm.at[pl.ds(wid * SP_W, SP_W)],
                    ttb_v.at[pl.ds(0, SP_W)])
    pltpu.sync_copy(pos_hbm.at[pl.ds(wid * SP_W, SP_W)],
                    ttb_v.at[pl.ds(SP_W, SP_W)])

    @plsc.parallel_loop(0, NBROWS * HS, unroll=8)
    def build(i):
        r = i // HS
        hs = pl.ds((i % HS) * 16, 16)
        ttb_v[r, hs] = ttb_v[r, hs] + ttbuf_v[r // SP_W, hs]

    lane = lax.iota(jnp.int32, 16)
    bufs = ((rows0_v, outb0_v, sem_w0, sem_o0),
            (rows1_v, outb1_v, sem_w1, sem_o1))

    def issue(ci, rows, sw):
        pltpu.async_copy(word_hbm.at[widx_v.at[ci]], rows, sw)

    def compute(ci, rows, outb):
        @plsc.parallel_loop(0, CHUNK)
        def p1(t):
            br = plsc.load_gather(brow_v, [jnp.full((16,), ci, jnp.int32),
                                           jnp.full((16,), t, jnp.int32)])
            z = jnp.zeros((16,), jnp.float32)

            @plsc.parallel_loop(0, HS, unroll=8, carry=(z, z))
            def accs(h, acc):
                acc_s, acc_q = acc
                hs = pl.ds(h * 16, 16)
                bslice = plsc.load_gather(ttb_v, [br, h * 16 + lane])
                v = rows[t, hs] + bslice
                rows[t, hs] = v
                return (acc_s + v, acc_q + v * v)

            acc_s, acc_q = accs
            sums_v[pl.ds(t * 16, 16)] = acc_s
            sq_v[pl.ds(t * 16, 16)] = acc_q

        col = lane * 16
        s_tot = jnp.zeros((16,), jnp.float32)
        q_tot = jnp.zeros((16,), jnp.float32)
        for l in range(16):
            s_tot = s_tot + plsc.load_gather(sums_v, [col + l])
            q_tot = q_tot + plsc.load_gather(sq_v, [col + l])
        mean = s_tot * (1.0 / HIDDEN)
        x = q_tot * (1.0 / HIDDEN) - mean * mean + LN_EPS
        iv = plsc.bitcast(x, jnp.int32)
        iv = 0x5F3759DF - lax.shift_right_logical(iv, 1)
        y = plsc.bitcast(iv, jnp.float32)
        xh = x * 0.5
        y = y * (1.5 - xh * y * y)
        y = y * (1.5 - xh * y * y)
        y = y * (1.5 - xh * y * y)
        a_v[pl.ds(0, CHUNK)] = y
        b_v[pl.ds(0, CHUNK)] = mean * y

        @plsc.parallel_loop(0, CHUNK // 8)
        def p3(g):
            t0 = g * 8
            ab = []
            for j in range(8):
                ti = jnp.full((16,), t0 + j, jnp.int32)
                ab.append((plsc.load_gather(a_v, [ti]),
                           plsc.load_gather(b_v, [ti])))

            @plsc.parallel_loop(0, HS, unroll=2)
            def apply(h):
                hs = pl.ds(h * 16, 16)
                gm = gam_v[hs]
                bt = bet_v[hs]
                for j in range(8):
                    a, b = ab[j]
                    outb[t0 + j, hs] = (rows[t0 + j, hs] * a - b) * gm + bt

    issue(0, rows0_v, sem_w0)
    issue(1, rows1_v, sem_w1)

    def pair(c, _):
        for k in (0, 1):
            rows, outb, sw, so = bufs[k]
            ci = 2 * c + k
            pltpu.make_async_copy(word_hbm.at[widx_v.at[ci]], rows, sw).wait()

            @pl.when(ci >= 2)
            def _():
                pltpu.make_async_copy(
                    outb, out_hbm.at[pl.ds(wid * SP_W, CHUNK)], so).wait()

            compute(ci, rows, outb)
            pltpu.async_copy(
                outb, out_hbm.at[pl.ds(ci * S_LEN + wid * SP_W, CHUNK)], so)

            @pl.when(ci + 2 < NCH)
            def _():
                issue(ci + 2, rows, sw)
        return 0

    lax.fori_loop(0, NCH // 2, pair, 0)

    pltpu.make_async_copy(outb0_v, out_hbm.at[pl.ds(0, CHUNK)], sem_o0).wait()
    pltpu.make_async_copy(outb1_v, out_hbm.at[pl.ds(0, CHUNK)], sem_o1).wait()


def kernel(input_ids, token_type_ids, word_emb, token_type_emb, pos_emb,
           ln_gamma, ln_beta):
    ids = input_ids.reshape(B_SZ, NW, SP_W).transpose(1, 0, 2)
    brow = (token_type_ids.reshape(B_SZ, NW, SP_W).transpose(1, 0, 2) * SP_W
            + jnp.arange(SP_W, dtype=jnp.int32)[None, None, :])
    out = _emb_layernorm(ids, brow, word_emb, token_type_emb,
                         pos_emb[:S_LEN], ln_gamma, ln_beta)
    return out.reshape(B_SZ, S_LEN, HIDDEN)
